# baseline (speedup 1.0000x reference)
import functools

import jax
import jax.numpy as jnp
from jax import lax
from jax.experimental import pallas as pl
from jax.experimental.pallas import tpu as pltpu
from jax.experimental.pallas import tpu_sc as plsc

N = 10000
E = 320000
H = 128
EPS = 1e-5

NC = 2
NS = 16
L = 16
NW = NC * NS
EPW = E // NW
B = 80
NCH = EPW // B
RPT = 624
ZROWS = 48
PK_SHIFT = 14


_sc_mesh = plsc.VectorSubcoreMesh(core_axis_name="c", subcore_axis_name="s")


@functools.partial(
    pl.kernel,
    mesh=_sc_mesh,
    out_type=jax.ShapeDtypeStruct((2 * N, H), jnp.float32),
    scratch_types=[
        pltpu.VMEM((NCH * B,), jnp.int32),
        pltpu.VMEM((B,), jnp.int32),
        pltpu.VMEM((B,), jnp.int32),
        pltpu.VMEM((B,), jnp.int32),
        pltpu.VMEM((B,), jnp.int32),
        pltpu.VMEM((B,), jnp.int32),
        pltpu.VMEM((B,), jnp.int32),
        pltpu.VMEM((B,), jnp.float32),
        pltpu.VMEM((B,), jnp.float32),
        pltpu.VMEM((B,), jnp.float32),
        pltpu.VMEM((B, H), jnp.float32),
        pltpu.VMEM((B, H), jnp.float32),
        pltpu.VMEM((B, H), jnp.float32),
        pltpu.VMEM((ZROWS, H), jnp.float32),
        pltpu.VMEM_SHARED((N, H), jnp.float32),
        pltpu.SemaphoreType.DMA,
        pltpu.SemaphoreType.DMA,
        pltpu.SemaphoreType.DMA,
        pltpu.SemaphoreType.DMA,
        pltpu.SemaphoreType.DMA,
        pltpu.SemaphoreType.DMA,
        pltpu.SemaphoreType.DMA,
        pltpu.SemaphoreType.DMA,
        pltpu.SemaphoreType.DMA,
        pltpu.SemaphoreType.DMA,
    ],
)
def _seg_sum_sc(y_hbm, pk_hbm, ew_hbm, out_hbm,
                pk_v, sc0, sc1, sc2, dc0, dc1, dc2, wc0, wc1, wc2,
                buf0, buf1, buf2, zero_v, acc,
                sem_pre, sg0, sg1, sg2, ss0, ss1, ss2, sw0, sw1, sw2):
    cid = lax.axis_index("c")
    sid = lax.axis_index("s")
    wid = cid * NS + sid
    src_c = (sc0, sc1, sc2)
    dst_c = (dc0, dc1, dc2)
    ew_c = (wc0, wc1, wc2)
    bufs = (buf0, buf1, buf2)
    sg = (sg0, sg1, sg2)
    ss = (ss0, ss1, ss2)
    sw = (sw0, sw1, sw2)

    pre_p = pltpu.async_copy(pk_hbm.at[wid], pk_v, sem_pre)

    zvec = jnp.zeros((L,), jnp.float32)

    def zrow(i, _):
        for f in range(H // L):
            zero_v[i, pl.ds(f * L, L)] = zvec
        return 0

    lax.fori_loop(0, ZROWS, zrow, 0)

    ew_row0 = wid * NCH

    def w_start(j, t):
        pltpu.async_copy(ew_hbm.at[ew_row0 + j], ew_c[t], sw[t])

    def w_wait(j, t):
        pltpu.make_async_copy(ew_hbm.at[ew_row0 + j], ew_c[t], sw[t]).wait()

    def unpack_src(j, t):
        for g in range(B // L):
            v = pk_v[pl.ds(j * B + g * L, L)]
            src_c[t][pl.ds(g * L, L)] = lax.shift_right_logical(v, PK_SHIFT)

    def unpack_dst(j, t):
        for g in range(B // L):
            v = pk_v[pl.ds(j * B + g * L, L)]
            dst_c[t][pl.ds(g * L, L)] = lax.bitwise_and(
                v, (1 << PK_SHIFT) - 1)

    def g_start(t):
        pltpu.async_copy(y_hbm.at[src_c[t]], bufs[t], sg[t])

    def g_wait(t):
        pltpu.make_async_copy(y_hbm.at[src_c[t]], bufs[t], sg[t]).wait()

    def s_start(t):
        pltpu.async_copy(bufs[t], acc.at[dst_c[t]], ss[t], add=True)

    def s_wait(t):
        pltpu.make_async_copy(bufs[t], acc.at[dst_c[t]], ss[t]).wait()

    def scale(j, t):
        for g in range(B // L):
            wch = ew_c[t][pl.ds(g * L, L)]
            for jj in range(L):
                w = wch[jj]
                r = g * L + jj
                for f in range(H // L):
                    bufs[t][r, pl.ds(f * L, L)] = (
                        bufs[t][r, pl.ds(f * L, L)] * w)

    def step(j, b):
        bp1 = (b + 1) % 3
        bp2 = (b + 2) % 3

        @pl.when(j >= 2)
        def _():
            s_wait(bp1)

        @pl.when(j + 1 < NCH)
        def _():
            unpack_src(j + 1, bp1)
            g_start(bp1)

        g_wait(b)
        w_wait(j, b)
        scale(j, b)
        s_start(b)

        @pl.when(j + 1 < NCH)
        def _():
            unpack_dst(j + 1, bp1)

        @pl.when(j + 2 < NCH)
        def _():
            w_start(j + 2, bp2)

    pre_p.wait()
    w_start(0, 0)
    w_start(1, 1)
    unpack_src(0, 0)
    unpack_dst(0, 0)
    g_start(0)

    def zcopy(k, _):
        pltpu.sync_copy(zero_v, acc.at[pl.ds(sid * RPT + k * ZROWS, ZROWS)])
        return 0

    lax.fori_loop(0, RPT // ZROWS, zcopy, 0)

    @pl.when(sid == NS - 1)
    def _():
        pltpu.sync_copy(zero_v.at[pl.ds(0, 16)], acc.at[pl.ds(NS * RPT, 16)])

    plsc.subcore_barrier()

    def triple(k, _):
        step(3 * k, 0)
        step(3 * k + 1, 1)
        step(3 * k + 2, 2)
        return 0

    lax.fori_loop(0, NCH // 3, triple, 0)
    for j in range(NCH - NCH % 3, NCH):
        step(j, j % 3)
    s_wait((NCH - 2) % 3)
    s_wait((NCH - 1) % 3)
    plsc.subcore_barrier()

    r0 = sid * RPT
    pltpu.sync_copy(acc.at[pl.ds(r0, RPT)],
                    out_hbm.at[pl.ds(cid * N + r0, RPT)])

    @pl.when(sid == NS - 1)
    def _():
        pltpu.sync_copy(acc.at[pl.ds(NS * RPT, 16)],
                        out_hbm.at[pl.ds(cid * N + NS * RPT, 16)])


def _seg_sum(y, pk3, ew2):
    return _seg_sum_sc(y, pk3, ew2)


RB = 2000
GRID = N // RB
HGRID = N // RB
_PREC = None


def _dot(a, b):
    return jnp.dot(a, b, preferred_element_type=jnp.float32,
                   precision=_PREC)


def _layer12_body(p0_ref, p1_ref, y_ref, wr_ref, wo_ref, c_ref, o_ref):
    agg = p0_ref[...] + p1_ref[...]
    t = _dot(agg, wr_ref[...]) + _dot(y_ref[...], wo_ref[...]) + c_ref[...]
    o_ref[...] = jnp.maximum(t, 0.0)


def _layer12(pp, y, wr, wo, c):
    hi, ho = wr.shape
    return pl.pallas_call(
        _layer12_body,
        grid=(GRID,),
        in_specs=[
            pl.BlockSpec((RB, H), lambda i: (i, 0)),
            pl.BlockSpec((RB, H), lambda i: (HGRID + i, 0)),
            pl.BlockSpec((RB, hi), lambda i: (i, 0)),
            pl.BlockSpec((hi, ho), lambda i: (0, 0)),
            pl.BlockSpec((hi, ho), lambda i: (0, 0)),
            pl.BlockSpec((1, ho), lambda i: (0, 0)),
        ],
        out_specs=pl.BlockSpec((RB, ho), lambda i: (i, 0)),
        out_shape=jax.ShapeDtypeStruct((N, ho), jnp.float32),
    )(pp, pp, y, wr, wo, c)


def _layer2z_body(p0_ref, p1_ref, y_ref, wr_ref, wo_ref, c_ref, w3r_ref,
                  h_ref, z_ref):
    agg = p0_ref[...] + p1_ref[...]
    t = _dot(agg, wr_ref[...]) + _dot(y_ref[...], wo_ref[...]) + c_ref[...]
    h = jnp.maximum(t, 0.0)
    h_ref[...] = h
    z_ref[...] = _dot(h, w3r_ref[...])


def _layer2z(pp, y, wr, wo, c, w3r):
    hi, ho = wr.shape
    return pl.pallas_call(
        _layer2z_body,
        grid=(GRID,),
        in_specs=[
            pl.BlockSpec((RB, H), lambda i: (i, 0)),
            pl.BlockSpec((RB, H), lambda i: (HGRID + i, 0)),
            pl.BlockSpec((RB, hi), lambda i: (i, 0)),
            pl.BlockSpec((hi, ho), lambda i: (0, 0)),
            pl.BlockSpec((hi, ho), lambda i: (0, 0)),
            pl.BlockSpec((1, ho), lambda i: (0, 0)),
            pl.BlockSpec((ho, H), lambda i: (0, 0)),
        ],
        out_specs=[
            pl.BlockSpec((RB, ho), lambda i: (i, 0)),
            pl.BlockSpec((RB, H), lambda i: (i, 0)),
        ],
        out_shape=[
            jax.ShapeDtypeStruct((N, ho), jnp.float32),
            jax.ShapeDtypeStruct((N, H), jnp.float32),
        ],
    )(pp, pp, y, wr, wo, c, w3r)


def _layer3_body(r0_ref, r1_ref, h2_ref, w3o_ref, c_ref, h1_ref,
                 wc1_ref, bc1_ref, wc2_ref, bc2_ref,
                 h3_ref, logits_ref, ge_ref, sum_ref, max_ref):
    i = pl.program_id(0)
    t = (r0_ref[...] + r1_ref[...]
         + _dot(h2_ref[...], w3o_ref[...])
         + c_ref[...] + h1_ref[...])
    h3 = jnp.maximum(t, 0.0)
    h3_ref[...] = h3
    bsum = jnp.sum(h3, axis=0, keepdims=True)
    bmax = jnp.max(h3, axis=0, keepdims=True)

    @pl.when(i == 0)
    def _():
        sum_ref[...] = bsum
        max_ref[...] = bmax

    @pl.when(i > 0)
    def _():
        sum_ref[...] = sum_ref[...] + bsum
        max_ref[...] = jnp.maximum(max_ref[...], bmax)

    @pl.when(i == GRID - 1)
    def _():
        ge = jnp.concatenate([sum_ref[...] * (1.0 / N), max_ref[...]], axis=1)
        ge_ref[...] = ge
        hid = jnp.maximum(_dot(ge, wc1_ref[...]) + bc1_ref[...], 0.0)
        logits_ref[...] = _dot(hid, wc2_ref[...]) + bc2_ref[...]


def _layer3(rr, h2, w3o, c, h1, wc1, bc1, wc2, bc2):
    return pl.pallas_call(
        _layer3_body,
        grid=(GRID,),
        in_specs=[
            pl.BlockSpec((RB, H), lambda i: (i, 0)),
            pl.BlockSpec((RB, H), lambda i: (HGRID + i, 0)),
            pl.BlockSpec((RB, 2 * H), lambda i: (i, 0)),
            pl.BlockSpec((2 * H, H), lambda i: (0, 0)),
            pl.BlockSpec((1, H), lambda i: (0, 0)),
            pl.BlockSpec((RB, H), lambda i: (i, 0)),
            pl.BlockSpec((2 * H, H), lambda i: (0, 0)),
            pl.BlockSpec((1, H), lambda i: (0, 0)),
            pl.BlockSpec((H, 2), lambda i: (0, 0)),
            pl.BlockSpec((1, 2), lambda i: (0, 0)),
        ],
        out_specs=[
            pl.BlockSpec((RB, H), lambda i: (i, 0)),
            pl.BlockSpec((1, 2), lambda i: (0, 0)),
            pl.BlockSpec((1, 2 * H), lambda i: (0, 0)),
        ],
        out_shape=[
            jax.ShapeDtypeStruct((N, H), jnp.float32),
            jax.ShapeDtypeStruct((1, 2), jnp.float32),
            jax.ShapeDtypeStruct((1, 2 * H), jnp.float32),
        ],
        scratch_shapes=[
            pltpu.VMEM((1, H), jnp.float32),
            pltpu.VMEM((1, H), jnp.float32),
        ],
    )(rr, rr, h2, w3o, c, h1, wc1, bc1, wc2, bc2)


def kernel(x, edge_index, edge_weight, W1_rel, b1, W1_root, W2_rel, b2,
           W2_root, W3_rel, b3, W3_root, g1, be1, g2, be2, g3, be3,
           Wc1, bc1, Wc2, bc2):
    src = edge_index[0].astype(jnp.int32)
    dst = edge_index[1].astype(jnp.int32)
    pk3 = ((src << PK_SHIFT) | dst).reshape(NW, NCH * B)
    ew2 = edge_weight.astype(jnp.float32).reshape(NW * NCH, B)

    s = 1.0 / jnp.sqrt(1.0 + EPS)
    a1 = s * g1
    a2 = s * g2
    a3 = s * g3
    W1r = W1_rel * a1[None, :]
    W1o = W1_root * a1[None, :]
    c1 = (b1 * a1 + be1)[None, :]
    W2r = W2_rel * a2[None, :]
    W2o = W2_root * a2[None, :]
    c2 = (b2 * a2 + be2)[None, :]
    W3r = W3_rel * a3[None, :]
    W3o = W3_root * a3[None, :]
    c3 = (b3 * a3 + be3)[None, :]

    pp = _seg_sum(x, pk3, ew2)
    h1 = _layer12(pp, x, W1r, W1o, c1)
    qq = _seg_sum(h1, pk3, ew2)
    h2, z2 = _layer2z(qq, h1, W2r, W2o, c2, W3r)
    rr = _seg_sum(z2, pk3, ew2)
    h3, logits, ge = _layer3(rr, h2, W3o, c3, h1,
                             Wc1, bc1[None, :], Wc2, bc2[None, :])
    return (logits, h3, ge)


# speedup vs baseline: 1.3357x; 1.3357x over previous
import functools

import jax
import jax.numpy as jnp
from jax import lax
from jax.experimental import pallas as pl
from jax.experimental.pallas import tpu as pltpu
from jax.experimental.pallas import tpu_sc as plsc

N = 10000
E = 320000
H = 128
EPS = 1e-5

NC = 2
NS = 16
L = 16
NW = NC * NS
EPW = E // NW
B = 80
NCH = EPW // B
RPT = 624
ZROWS = 48
PK_SHIFT = 14


_sc_mesh = plsc.VectorSubcoreMesh(core_axis_name="c", subcore_axis_name="s")


@functools.partial(
    pl.kernel,
    mesh=_sc_mesh,
    out_type=jax.ShapeDtypeStruct((2 * N, H), jnp.float32),
    scratch_types=[
        pltpu.VMEM((NCH * B,), jnp.int32),
        pltpu.VMEM((B,), jnp.int32),
        pltpu.VMEM((B,), jnp.int32),
        pltpu.VMEM((B,), jnp.int32),
        pltpu.VMEM((B,), jnp.int32),
        pltpu.VMEM((B,), jnp.int32),
        pltpu.VMEM((B,), jnp.int32),
        pltpu.VMEM((B,), jnp.float32),
        pltpu.VMEM((B,), jnp.float32),
        pltpu.VMEM((B,), jnp.float32),
        pltpu.VMEM((B, H), jnp.float32),
        pltpu.VMEM((B, H), jnp.float32),
        pltpu.VMEM((B, H), jnp.float32),
        pltpu.VMEM((ZROWS, H), jnp.float32),
        pltpu.VMEM_SHARED((N, H), jnp.float32),
        pltpu.SemaphoreType.DMA,
        pltpu.SemaphoreType.DMA,
        pltpu.SemaphoreType.DMA,
        pltpu.SemaphoreType.DMA,
        pltpu.SemaphoreType.DMA,
        pltpu.SemaphoreType.DMA,
        pltpu.SemaphoreType.DMA,
        pltpu.SemaphoreType.DMA,
        pltpu.SemaphoreType.DMA,
        pltpu.SemaphoreType.DMA,
    ],
)
def _seg_sum_sc(y_hbm, pk_hbm, ew_hbm, out_hbm,
                pk_v, sc0, sc1, sc2, dc0, dc1, dc2, wc0, wc1, wc2,
                buf0, buf1, buf2, zero_v, acc,
                sem_pre, sg0, sg1, sg2, ss0, ss1, ss2, sw0, sw1, sw2):
    cid = lax.axis_index("c")
    sid = lax.axis_index("s")
    wid = cid * NS + sid
    src_c = (sc0, sc1, sc2)
    dst_c = (dc0, dc1, dc2)
    ew_c = (wc0, wc1, wc2)
    bufs = (buf0, buf1, buf2)
    sg = (sg0, sg1, sg2)
    ss = (ss0, ss1, ss2)
    sw = (sw0, sw1, sw2)

    pre_p = pltpu.async_copy(pk_hbm.at[wid], pk_v, sem_pre)

    zvec = jnp.zeros((L,), jnp.float32)

    def zrow(i, _):
        for f in range(H // L):
            zero_v[i, pl.ds(f * L, L)] = zvec
        return 0

    lax.fori_loop(0, ZROWS, zrow, 0)

    ew_row0 = wid * NCH

    def w_start(j, t):
        pltpu.async_copy(ew_hbm.at[ew_row0 + j], ew_c[t], sw[t])

    def w_wait(j, t):
        pltpu.make_async_copy(ew_hbm.at[ew_row0 + j], ew_c[t], sw[t]).wait()

    def unpack_src(j, t):
        for g in range(B // L):
            v = pk_v[pl.ds(j * B + g * L, L)]
            src_c[t][pl.ds(g * L, L)] = lax.shift_right_logical(v, PK_SHIFT)

    def unpack_dst(j, t):
        for g in range(B // L):
            v = pk_v[pl.ds(j * B + g * L, L)]
            dst_c[t][pl.ds(g * L, L)] = lax.bitwise_and(
                v, (1 << PK_SHIFT) - 1)

    def g_start(t):
        pltpu.async_copy(y_hbm.at[src_c[t]], bufs[t], sg[t])

    def g_wait(t):
        pltpu.make_async_copy(y_hbm.at[src_c[t]], bufs[t], sg[t]).wait()

    def s_start(t):
        pltpu.async_copy(bufs[t], acc.at[dst_c[t]], ss[t], add=True)

    def s_wait(t):
        pltpu.make_async_copy(bufs[t], acc.at[dst_c[t]], ss[t]).wait()

    def scale(j, t):
        def grp(g, _):
            wch = ew_c[t][pl.ds(g * L, L)]
            for jj in range(L):
                w = wch[jj]
                r = g * L + jj
                for f in range(H // L):
                    bufs[t][r, pl.ds(f * L, L)] = (
                        bufs[t][r, pl.ds(f * L, L)] * w)
            return 0

        lax.fori_loop(0, B // L, grp, 0)

    def step(j, b):
        bp1 = (b + 1) % 3
        bp2 = (b + 2) % 3

        @pl.when(j >= 2)
        def _():
            s_wait(bp1)

        @pl.when(j + 1 < NCH)
        def _():
            unpack_src(j + 1, bp1)
            g_start(bp1)

        w_wait(j, b)
        g_wait(b)
        scale(j, b)
        s_start(b)

        @pl.when(j + 1 < NCH)
        def _():
            unpack_dst(j + 1, bp1)

        @pl.when(j + 2 < NCH)
        def _():
            w_start(j + 2, bp2)

    pre_p.wait()
    w_start(0, 0)
    w_start(1, 1)
    unpack_src(0, 0)
    unpack_dst(0, 0)
    g_start(0)

    def zcopy(k, _):
        pltpu.async_copy(zero_v,
                         acc.at[pl.ds(sid * RPT + k * ZROWS, ZROWS)], sem_pre)
        return 0

    lax.fori_loop(0, RPT // ZROWS, zcopy, 0)

    @pl.when(sid == NS - 1)
    def _():
        pltpu.sync_copy(zero_v.at[pl.ds(0, 16)], acc.at[pl.ds(NS * RPT, 16)])

    def zdrain(k, _):
        pltpu.make_async_copy(
            zero_v, acc.at[pl.ds(sid * RPT + k * ZROWS, ZROWS)],
            sem_pre).wait()
        return 0

    lax.fori_loop(0, RPT // ZROWS, zdrain, 0)
    plsc.subcore_barrier()

    def triple(k, _):
        step(3 * k, 0)
        step(3 * k + 1, 1)
        step(3 * k + 2, 2)
        return 0

    lax.fori_loop(0, NCH // 3, triple, 0)
    for j in range(NCH - NCH % 3, NCH):
        step(j, j % 3)
    s_wait((NCH - 2) % 3)
    s_wait((NCH - 1) % 3)
    plsc.subcore_barrier()

    r0 = sid * RPT
    pltpu.sync_copy(acc.at[pl.ds(r0, RPT)],
                    out_hbm.at[pl.ds(cid * N + r0, RPT)])

    @pl.when(sid == NS - 1)
    def _():
        pltpu.sync_copy(acc.at[pl.ds(NS * RPT, 16)],
                        out_hbm.at[pl.ds(cid * N + NS * RPT, 16)])


def _seg_sum(y, pk3, ew2):
    return _seg_sum_sc(y, pk3, ew2)


RB = 2000
GRID = N // RB
HGRID = N // RB
_PREC = None


def _dot(a, b):
    return jnp.dot(a, b, preferred_element_type=jnp.float32,
                   precision=_PREC)


def _layer12_body(p0_ref, p1_ref, y_ref, wr_ref, wo_ref, c_ref, o_ref):
    agg = p0_ref[...] + p1_ref[...]
    t = _dot(agg, wr_ref[...]) + _dot(y_ref[...], wo_ref[...]) + c_ref[...]
    o_ref[...] = jnp.maximum(t, 0.0)


def _layer12(pp, y, wr, wo, c):
    hi, ho = wr.shape
    return pl.pallas_call(
        _layer12_body,
        grid=(GRID,),
        in_specs=[
            pl.BlockSpec((RB, H), lambda i: (i, 0)),
            pl.BlockSpec((RB, H), lambda i: (HGRID + i, 0)),
            pl.BlockSpec((RB, hi), lambda i: (i, 0)),
            pl.BlockSpec((hi, ho), lambda i: (0, 0)),
            pl.BlockSpec((hi, ho), lambda i: (0, 0)),
            pl.BlockSpec((1, ho), lambda i: (0, 0)),
        ],
        out_specs=pl.BlockSpec((RB, ho), lambda i: (i, 0)),
        out_shape=jax.ShapeDtypeStruct((N, ho), jnp.float32),
    )(pp, pp, y, wr, wo, c)


def _layer2z_body(p0_ref, p1_ref, y_ref, wr_ref, wo_ref, c_ref, w3r_ref,
                  h_ref, z_ref):
    agg = p0_ref[...] + p1_ref[...]
    t = _dot(agg, wr_ref[...]) + _dot(y_ref[...], wo_ref[...]) + c_ref[...]
    h = jnp.maximum(t, 0.0)
    h_ref[...] = h
    z_ref[...] = _dot(h, w3r_ref[...])


def _layer2z(pp, y, wr, wo, c, w3r):
    hi, ho = wr.shape
    return pl.pallas_call(
        _layer2z_body,
        grid=(GRID,),
        in_specs=[
            pl.BlockSpec((RB, H), lambda i: (i, 0)),
            pl.BlockSpec((RB, H), lambda i: (HGRID + i, 0)),
            pl.BlockSpec((RB, hi), lambda i: (i, 0)),
            pl.BlockSpec((hi, ho), lambda i: (0, 0)),
            pl.BlockSpec((hi, ho), lambda i: (0, 0)),
            pl.BlockSpec((1, ho), lambda i: (0, 0)),
            pl.BlockSpec((ho, H), lambda i: (0, 0)),
        ],
        out_specs=[
            pl.BlockSpec((RB, ho), lambda i: (i, 0)),
            pl.BlockSpec((RB, H), lambda i: (i, 0)),
        ],
        out_shape=[
            jax.ShapeDtypeStruct((N, ho), jnp.float32),
            jax.ShapeDtypeStruct((N, H), jnp.float32),
        ],
    )(pp, pp, y, wr, wo, c, w3r)


def _layer3_body(r0_ref, r1_ref, h2_ref, w3o_ref, c_ref, h1_ref,
                 wc1_ref, bc1_ref, wc2_ref, bc2_ref,
                 h3_ref, logits_ref, ge_ref, sum_ref, max_ref):
    i = pl.program_id(0)
    t = (r0_ref[...] + r1_ref[...]
         + _dot(h2_ref[...], w3o_ref[...])
         + c_ref[...] + h1_ref[...])
    h3 = jnp.maximum(t, 0.0)
    h3_ref[...] = h3
    bsum = jnp.sum(h3, axis=0, keepdims=True)
    bmax = jnp.max(h3, axis=0, keepdims=True)

    @pl.when(i == 0)
    def _():
        sum_ref[...] = bsum
        max_ref[...] = bmax

    @pl.when(i > 0)
    def _():
        sum_ref[...] = sum_ref[...] + bsum
        max_ref[...] = jnp.maximum(max_ref[...], bmax)

    @pl.when(i == GRID - 1)
    def _():
        ge = jnp.concatenate([sum_ref[...] * (1.0 / N), max_ref[...]], axis=1)
        ge_ref[...] = ge
        hid = jnp.maximum(_dot(ge, wc1_ref[...]) + bc1_ref[...], 0.0)
        logits_ref[...] = _dot(hid, wc2_ref[...]) + bc2_ref[...]


def _layer3(rr, h2, w3o, c, h1, wc1, bc1, wc2, bc2):
    return pl.pallas_call(
        _layer3_body,
        grid=(GRID,),
        in_specs=[
            pl.BlockSpec((RB, H), lambda i: (i, 0)),
            pl.BlockSpec((RB, H), lambda i: (HGRID + i, 0)),
            pl.BlockSpec((RB, 2 * H), lambda i: (i, 0)),
            pl.BlockSpec((2 * H, H), lambda i: (0, 0)),
            pl.BlockSpec((1, H), lambda i: (0, 0)),
            pl.BlockSpec((RB, H), lambda i: (i, 0)),
            pl.BlockSpec((2 * H, H), lambda i: (0, 0)),
            pl.BlockSpec((1, H), lambda i: (0, 0)),
            pl.BlockSpec((H, 2), lambda i: (0, 0)),
            pl.BlockSpec((1, 2), lambda i: (0, 0)),
        ],
        out_specs=[
            pl.BlockSpec((RB, H), lambda i: (i, 0)),
            pl.BlockSpec((1, 2), lambda i: (0, 0)),
            pl.BlockSpec((1, 2 * H), lambda i: (0, 0)),
        ],
        out_shape=[
            jax.ShapeDtypeStruct((N, H), jnp.float32),
            jax.ShapeDtypeStruct((1, 2), jnp.float32),
            jax.ShapeDtypeStruct((1, 2 * H), jnp.float32),
        ],
        scratch_shapes=[
            pltpu.VMEM((1, H), jnp.float32),
            pltpu.VMEM((1, H), jnp.float32),
        ],
    )(rr, rr, h2, w3o, c, h1, wc1, bc1, wc2, bc2)


def kernel(x, edge_index, edge_weight, W1_rel, b1, W1_root, W2_rel, b2,
           W2_root, W3_rel, b3, W3_root, g1, be1, g2, be2, g3, be3,
           Wc1, bc1, Wc2, bc2):
    src = edge_index[0].astype(jnp.int32)
    dst = edge_index[1].astype(jnp.int32)
    pk3 = ((src << PK_SHIFT) | dst).reshape(NW, NCH * B)
    ew2 = edge_weight.astype(jnp.float32).reshape(NW * NCH, B)

    s = 1.0 / jnp.sqrt(1.0 + EPS)
    a1 = s * g1
    a2 = s * g2
    a3 = s * g3
    W1r = W1_rel * a1[None, :]
    W1o = W1_root * a1[None, :]
    c1 = (b1 * a1 + be1)[None, :]
    W2r = W2_rel * a2[None, :]
    W2o = W2_root * a2[None, :]
    c2 = (b2 * a2 + be2)[None, :]
    W3r = W3_rel * a3[None, :]
    W3o = W3_root * a3[None, :]
    c3 = (b3 * a3 + be3)[None, :]

    pp = _seg_sum(x, pk3, ew2)
    h1 = _layer12(pp, x, W1r, W1o, c1)
    qq = _seg_sum(h1, pk3, ew2)
    h2, z2 = _layer2z(qq, h1, W2r, W2o, c2, W3r)
    rr = _seg_sum(z2, pk3, ew2)
    h3, logits, ge = _layer3(rr, h2, W3o, c3, h1,
                             Wc1, bc1[None, :], Wc2, bc2[None, :])
    return (logits, h3, ge)
